# baseline (speedup 1.0000x reference)
import functools

import jax
import jax.numpy as jnp
from jax import lax
from jax.experimental import pallas as pl
from jax.experimental.pallas import tpu as pltpu
from jax.experimental.pallas import tpu_sc as plsc

N = 10000
E = 50000
DF = 128
DE = 16
H1 = 128
H2 = 64
C = 10
CP = 16

NC = 2
NS = 16
NW = NC * NS

N_PAD = 10240
E_PAD = 50176
E_HALF = E_PAD // 2
EPW = E_HALF // NW
GCHUNKS = 7
GCHUNK = EPW // GCHUNKS
SCHUNKS = 7
SCHUNK = EPW // SCHUNKS
ROW_STRIPE = N_PAD // NS

BF = jnp.bfloat16


def _sc_mesh():
    return plsc.VectorSubcoreMesh(core_axis_name="c", subcore_axis_name="s")


def _gather_body(table, idx, out, idx0, idx1, rows0, rows1,
                 si0, si1, sg, so0, so1):
    wid = lax.axis_index("s") * NC + lax.axis_index("c")
    base = wid * EPW
    idxb, rowb = [idx0, idx1], [rows0, rows1]
    sib, sob = [si0, si1], [so0, so1]
    idx_d = [None, None]
    out_d = [None, None]
    idx_d[0] = pltpu.async_copy(idx.at[pl.ds(base, GCHUNK)], idx0, si0)
    for c in range(GCHUNKS):
        sl, nx = c % 2, (c + 1) % 2
        if c + 1 < GCHUNKS:
            off = base + (c + 1) * GCHUNK
            idx_d[nx] = pltpu.async_copy(idx.at[pl.ds(off, GCHUNK)],
                                         idxb[nx], sib[nx])
        idx_d[sl].wait()
        if c >= 2:
            out_d[sl].wait()
        pltpu.async_copy(table.at[idxb[sl]], rowb[sl], sg).wait()
        out_d[sl] = pltpu.async_copy(
            rowb[sl], out.at[pl.ds(base + c * GCHUNK, GCHUNK)], sob[sl])
    out_d[0].wait()
    out_d[1].wait()


def _sc_gather(table, idx):
    return pl.kernel(
        _gather_body,
        out_type=jax.ShapeDtypeStruct((E_HALF, DF), jnp.float32),
        mesh=_sc_mesh(),
        scratch_types=[
            pltpu.VMEM((GCHUNK,), jnp.int32),
            pltpu.VMEM((GCHUNK,), jnp.int32),
            pltpu.VMEM((GCHUNK, DF), jnp.float32),
            pltpu.VMEM((GCHUNK, DF), jnp.float32),
            pltpu.SemaphoreType.DMA,
            pltpu.SemaphoreType.DMA,
            pltpu.SemaphoreType.DMA,
            pltpu.SemaphoreType.DMA,
            pltpu.SemaphoreType.DMA,
        ],
    )(table, idx)


def _scatter_body(msg, idx, zeros, out, idx0, idx1, rows0, rows1, acc,
                  sz, si0, si1, sm0, sm1):
    cid = lax.axis_index("c")
    sid = lax.axis_index("s")
    base = (sid * NC + cid) * EPW
    stripe = sid * ROW_STRIPE
    zd = pltpu.async_copy(zeros.at[pl.ds(stripe, ROW_STRIPE)],
                          acc.at[pl.ds(stripe, ROW_STRIPE)], sz)
    idxb, rowb = [idx0, idx1], [rows0, rows1]
    sib, smb = [si0, si1], [sm0, sm1]
    idx_d = [None, None]
    msg_d = [None, None]
    idx_d[0] = pltpu.async_copy(idx.at[pl.ds(base, SCHUNK)], idx0, si0)
    msg_d[0] = pltpu.async_copy(msg.at[pl.ds(base, SCHUNK)], rows0, sm0)
    zd.wait()
    plsc.subcore_barrier()
    for c in range(SCHUNKS):
        sl, nx = c % 2, (c + 1) % 2
        if c + 1 < SCHUNKS:
            off = base + (c + 1) * SCHUNK
            idx_d[nx] = pltpu.async_copy(idx.at[pl.ds(off, SCHUNK)],
                                         idxb[nx], sib[nx])
            msg_d[nx] = pltpu.async_copy(msg.at[pl.ds(off, SCHUNK)],
                                         rowb[nx], smb[nx])
        idx_d[sl].wait()
        msg_d[sl].wait()
        pltpu.sync_copy(rowb[sl], acc.at[idxb[sl]], add=True)
    plsc.subcore_barrier()
    pltpu.sync_copy(acc.at[pl.ds(stripe, ROW_STRIPE)],
                    out.at[cid, pl.ds(stripe, ROW_STRIPE)])


def _sc_scatter(msg, idx, zeros):
    return pl.kernel(
        _scatter_body,
        out_type=jax.ShapeDtypeStruct((NC, N_PAD, DF), jnp.float32),
        mesh=_sc_mesh(),
        scratch_types=[
            pltpu.VMEM((SCHUNK,), jnp.int32),
            pltpu.VMEM((SCHUNK,), jnp.int32),
            pltpu.VMEM((SCHUNK, DF), jnp.float32),
            pltpu.VMEM((SCHUNK, DF), jnp.float32),
            pltpu.VMEM_SHARED((N_PAD, DF), jnp.float32),
            pltpu.SemaphoreType.DMA,
            pltpu.SemaphoreType.DMA,
            pltpu.SemaphoreType.DMA,
            pltpu.SemaphoreType.DMA,
            pltpu.SemaphoreType.DMA,
        ],
    )(msg, idx, zeros)


BN = 2000
TE = 1792
NB_H = E_HALF // TE


def _stage_a_body(x_ref, w1_ref, b1_ref, r1_ref, bs1_ref, hroot_ref, xe_ref):
    x = x_ref[...]
    xe_ref[...] = x
    h = jnp.maximum(
        jnp.dot(x, w1_ref[...], preferred_element_type=jnp.float32)
        + b1_ref[...], 0.0)
    hroot_ref[...] = (
        jnp.dot(h, r1_ref[...], preferred_element_type=jnp.float32) + bs1_ref[...])


def _edge_mm_body(lin1, dr, half, xj_ref, ea_ref, w_ref, lw_ref, lb_ref, out_ref):
    i = pl.program_id(0)
    if lin1:
        xr = xj_ref[...].astype(BF)
        h = jnp.maximum(
            jnp.dot(xr, lw_ref[...], preferred_element_type=jnp.float32)
            + lb_ref[...], 0.0)
        xj = h.astype(BF)
    else:
        xj = xj_ref[...].astype(BF)
    parts = []
    for k in range(DE):
        bc = jnp.broadcast_to(ea_ref[:, k:k + 1].astype(BF), (TE, DF))
        parts.append(bc * xj)
    parts.append(xj)
    z = jnp.concatenate(parts, axis=1)
    acc = jnp.dot(z, w_ref[...], preferred_element_type=jnp.float32)
    rows = half * E_HALF + i * TE + lax.broadcasted_iota(jnp.int32, (TE, 1), 0)
    acc = jnp.where(rows < E, acc, 0.0)
    out_ref[...] = jnp.concatenate(
        [acc, jnp.zeros((TE, DF - dr), jnp.float32)], axis=1)


def _edge_mm(xj, ea, wbig, lw, lb, lin1, dr, half):
    def ea_map(i, h=half):
        return (i + h * NB_H, 0)
    return pl.pallas_call(
        functools.partial(_edge_mm_body, lin1, dr, half),
        grid=(NB_H,),
        in_specs=[
            pl.BlockSpec((TE, DF), lambda i: (i, 0)),
            pl.BlockSpec((TE, DE), ea_map),
            pl.BlockSpec(((DE + 1) * DF, dr), lambda i: (0, 0)),
            pl.BlockSpec((DF, DF), lambda i: (0, 0)),
            pl.BlockSpec((1, DF), lambda i: (0, 0)),
        ],
        out_specs=pl.BlockSpec((TE, DF), lambda i: (i, 0)),
        out_shape=jax.ShapeDtypeStruct((E_HALF, DF), jnp.float32),
    )(xj, ea, wbig, lw, lb)


def _combine1_body(aggA_ref, aggB_ref, hroot_ref, r2_ref, bs2_ref,
                   h2_ref, hroot2_ref):
    s = (aggA_ref[0][:, :H2] + aggA_ref[1][:, :H2]
         + aggB_ref[0][:, :H2] + aggB_ref[1][:, :H2] + hroot_ref[...])
    h2 = jnp.maximum(s, 0.0)
    h2_ref[...] = jnp.concatenate(
        [h2, jnp.zeros((BN, DF - H2), jnp.float32)], axis=1)
    hroot2_ref[...] = (
        jnp.dot(h2, r2_ref[...], preferred_element_type=jnp.float32) + bs2_ref[...])


def _final_body(agg2A_ref, agg2B_ref, hroot2_ref, out_ref):
    v = (agg2A_ref[0][:, :CP] + agg2A_ref[1][:, :CP]
         + agg2B_ref[0][:, :CP] + agg2B_ref[1][:, :CP] + hroot2_ref[...])
    col = lax.broadcasted_iota(jnp.int32, v.shape, 1)
    mask = col < C
    vm = jnp.where(mask, v, -1e30)
    m = jnp.max(vm, axis=1, keepdims=True)
    ex = jnp.where(mask, jnp.exp(v - m), 0.0)
    s = jnp.sum(ex, axis=1, keepdims=True)
    out_ref[...] = (v - m - jnp.log(s))[:, :C]


def kernel(x, edge_index, edge_attr, W1, b1, We1, be1, root1, bias1,
           We2, be2, root2, bias2):
    f32 = jnp.float32
    src_p = jnp.zeros((E_PAD,), jnp.int32).at[:E].set(edge_index[0])
    dst_p = jnp.zeros((E_PAD,), jnp.int32).at[:E].set(edge_index[1])
    srcA, srcB = src_p[:E_HALF], src_p[E_HALF:]
    dstA, dstB = dst_p[:E_HALF], dst_p[E_HALF:]
    ea_f = edge_attr

    wbig1 = jnp.concatenate(
        [We1.reshape(DE * H1, H2), be1.reshape(H1, H2)], axis=0).astype(BF)
    w2r = jnp.pad(We2.reshape(DE, H2, C), ((0, 0), (0, DF - H2), (0, CP - C)))
    be2r = jnp.pad(be2.reshape(H2, C), ((0, DF - H2), (0, CP - C)))
    wbig2 = jnp.concatenate(
        [w2r.reshape(DE * DF, CP), be2r], axis=0).astype(BF)
    w1_bf = W1.astype(BF)
    b1_row = b1.reshape(1, H1)
    root2p = jnp.pad(root2, ((0, 0), (0, CP - C)))
    bias2p = jnp.pad(bias2, (0, CP - C)).reshape(1, CP)
    zeros_acc = jnp.zeros((N_PAD, DF), f32)

    grid_n = (N // BN,)
    hroot, x_echo = pl.pallas_call(
        _stage_a_body,
        grid=grid_n,
        in_specs=[
            pl.BlockSpec((BN, DF), lambda i: (i, 0)),
            pl.BlockSpec((DF, H1), lambda i: (0, 0)),
            pl.BlockSpec((1, H1), lambda i: (0, 0)),
            pl.BlockSpec((H1, H2), lambda i: (0, 0)),
            pl.BlockSpec((1, H2), lambda i: (0, 0)),
        ],
        out_specs=[
            pl.BlockSpec((BN, H2), lambda i: (i, 0)),
            pl.BlockSpec((BN, DF), lambda i: (i, 0)),
        ],
        out_shape=[
            jax.ShapeDtypeStruct((N, H2), f32),
            jax.ShapeDtypeStruct((N, DF), f32),
        ],
    )(x, W1, b1_row, root1, bias1.reshape(1, H2))

    xrA = _sc_gather(x_echo, srcA)
    xrB = _sc_gather(x_echo, srcB)
    msgA = _edge_mm(xrA, ea_f, wbig1, w1_bf, b1_row, True, H2, 0)
    msgB = _edge_mm(xrB, ea_f, wbig1, w1_bf, b1_row, True, H2, 1)
    aggA = _sc_scatter(msgA, dstA, zeros_acc)
    aggB = _sc_scatter(msgB, dstB, zeros_acc)

    h2, hroot2 = pl.pallas_call(
        _combine1_body,
        grid=grid_n,
        in_specs=[
            pl.BlockSpec((NC, BN, DF), lambda i: (0, i, 0)),
            pl.BlockSpec((NC, BN, DF), lambda i: (0, i, 0)),
            pl.BlockSpec((BN, H2), lambda i: (i, 0)),
            pl.BlockSpec((H2, CP), lambda i: (0, 0)),
            pl.BlockSpec((1, CP), lambda i: (0, 0)),
        ],
        out_specs=[
            pl.BlockSpec((BN, DF), lambda i: (i, 0)),
            pl.BlockSpec((BN, CP), lambda i: (i, 0)),
        ],
        out_shape=[
            jax.ShapeDtypeStruct((N, DF), f32),
            jax.ShapeDtypeStruct((N, CP), f32),
        ],
    )(aggA, aggB, hroot, root2p, bias2p)

    xj2A = _sc_gather(h2, srcA)
    xj2B = _sc_gather(h2, srcB)
    msg2A = _edge_mm(xj2A, ea_f, wbig2, w1_bf, b1_row, False, CP, 0)
    msg2B = _edge_mm(xj2B, ea_f, wbig2, w1_bf, b1_row, False, CP, 1)
    agg2A = _sc_scatter(msg2A, dstA, zeros_acc)
    agg2B = _sc_scatter(msg2B, dstB, zeros_acc)

    out = pl.pallas_call(
        _final_body,
        grid=grid_n,
        in_specs=[
            pl.BlockSpec((NC, BN, DF), lambda i: (0, i, 0)),
            pl.BlockSpec((NC, BN, DF), lambda i: (0, i, 0)),
            pl.BlockSpec((BN, CP), lambda i: (i, 0)),
        ],
        out_specs=pl.BlockSpec((BN, C), lambda i: (i, 0)),
        out_shape=jax.ShapeDtypeStruct((N, C), f32),
    )(agg2A, agg2B, hroot2)

    return out


# speedup vs baseline: 1.0263x; 1.0263x over previous
import functools

import jax
import jax.numpy as jnp
from jax import lax
from jax.experimental import pallas as pl
from jax.experimental.pallas import tpu as pltpu
from jax.experimental.pallas import tpu_sc as plsc

N = 10000
E = 50000
DF = 128
DE = 16
H1 = 128
H2 = 64
C = 10
CP = 16

NC = 2
NS = 16
NW = NC * NS

N_PAD = 10240
E_PAD = 50176
E_HALF = E_PAD // 2
EPW = E_HALF // NW
GCHUNKS = 7
GCHUNK = EPW // GCHUNKS
SCHUNKS = 7
SCHUNK = EPW // SCHUNKS
ROW_STRIPE = N_PAD // NS

BF = jnp.bfloat16


def _sc_mesh():
    return plsc.VectorSubcoreMesh(core_axis_name="c", subcore_axis_name="s")


def _gather_body(table, idx, out, idx0, idx1, rows0, rows1,
                 si0, si1, sg, so0, so1):
    wid = lax.axis_index("s") * NC + lax.axis_index("c")
    base = wid * EPW
    idxb, rowb = [idx0, idx1], [rows0, rows1]
    sib, sob = [si0, si1], [so0, so1]
    idx_d = [None, None]
    out_d = [None, None]
    idx_d[0] = pltpu.async_copy(idx.at[pl.ds(base, GCHUNK)], idx0, si0)
    for c in range(GCHUNKS):
        sl, nx = c % 2, (c + 1) % 2
        if c + 1 < GCHUNKS:
            off = base + (c + 1) * GCHUNK
            idx_d[nx] = pltpu.async_copy(idx.at[pl.ds(off, GCHUNK)],
                                         idxb[nx], sib[nx])
        idx_d[sl].wait()
        if c >= 2:
            out_d[sl].wait()
        pltpu.async_copy(table.at[idxb[sl]], rowb[sl], sg).wait()
        out_d[sl] = pltpu.async_copy(
            rowb[sl], out.at[pl.ds(base + c * GCHUNK, GCHUNK)], sob[sl])
    out_d[0].wait()
    out_d[1].wait()


def _sc_gather(table, idx):
    return pl.kernel(
        _gather_body,
        out_type=jax.ShapeDtypeStruct((E_HALF, DF), jnp.float32),
        mesh=_sc_mesh(),
        scratch_types=[
            pltpu.VMEM((GCHUNK,), jnp.int32),
            pltpu.VMEM((GCHUNK,), jnp.int32),
            pltpu.VMEM((GCHUNK, DF), jnp.float32),
            pltpu.VMEM((GCHUNK, DF), jnp.float32),
            pltpu.SemaphoreType.DMA,
            pltpu.SemaphoreType.DMA,
            pltpu.SemaphoreType.DMA,
            pltpu.SemaphoreType.DMA,
            pltpu.SemaphoreType.DMA,
        ],
    )(table, idx)


def _scatter_body(msg, idx, zeros, out, idx0, idx1, rows0, rows1, acc,
                  sz, si0, si1, sm0, sm1):
    cid = lax.axis_index("c")
    sid = lax.axis_index("s")
    base = (sid * NC + cid) * EPW
    stripe = sid * ROW_STRIPE
    zd = pltpu.async_copy(zeros.at[pl.ds(stripe, ROW_STRIPE)],
                          acc.at[pl.ds(stripe, ROW_STRIPE)], sz)
    idxb, rowb = [idx0, idx1], [rows0, rows1]
    sib, smb = [si0, si1], [sm0, sm1]
    idx_d = [None, None]
    msg_d = [None, None]
    idx_d[0] = pltpu.async_copy(idx.at[pl.ds(base, SCHUNK)], idx0, si0)
    msg_d[0] = pltpu.async_copy(msg.at[pl.ds(base, SCHUNK)], rows0, sm0)
    zd.wait()
    plsc.subcore_barrier()
    for c in range(SCHUNKS):
        sl, nx = c % 2, (c + 1) % 2
        if c + 1 < SCHUNKS:
            off = base + (c + 1) * SCHUNK
            idx_d[nx] = pltpu.async_copy(idx.at[pl.ds(off, SCHUNK)],
                                         idxb[nx], sib[nx])
            msg_d[nx] = pltpu.async_copy(msg.at[pl.ds(off, SCHUNK)],
                                         rowb[nx], smb[nx])
        idx_d[sl].wait()
        msg_d[sl].wait()
        pltpu.sync_copy(rowb[sl], acc.at[idxb[sl]], add=True)
    plsc.subcore_barrier()
    pltpu.sync_copy(acc.at[pl.ds(stripe, ROW_STRIPE)],
                    out.at[cid, pl.ds(stripe, ROW_STRIPE)])


def _sc_scatter(msg, idx, zeros):
    return pl.kernel(
        _scatter_body,
        out_type=jax.ShapeDtypeStruct((NC, N_PAD, DF), jnp.float32),
        mesh=_sc_mesh(),
        scratch_types=[
            pltpu.VMEM((SCHUNK,), jnp.int32),
            pltpu.VMEM((SCHUNK,), jnp.int32),
            pltpu.VMEM((SCHUNK, DF), jnp.float32),
            pltpu.VMEM((SCHUNK, DF), jnp.float32),
            pltpu.VMEM_SHARED((N_PAD, DF), jnp.float32),
            pltpu.SemaphoreType.DMA,
            pltpu.SemaphoreType.DMA,
            pltpu.SemaphoreType.DMA,
            pltpu.SemaphoreType.DMA,
            pltpu.SemaphoreType.DMA,
        ],
    )(msg, idx, zeros)


BN = 2000
TE = 3136
NB_H = E_HALF // TE


def _stage_a_body(x_ref, w1_ref, b1_ref, r1_ref, bs1_ref, hroot_ref):
    h = jnp.maximum(
        jnp.dot(x_ref[...], w1_ref[...], preferred_element_type=jnp.float32)
        + b1_ref[...], 0.0)
    hroot_ref[...] = (
        jnp.dot(h, r1_ref[...], preferred_element_type=jnp.float32) + bs1_ref[...])


def _edge_mm_body(lin1, dr, half, xj_ref, ea_ref, w_ref, lw_ref, lb_ref, out_ref):
    i = pl.program_id(0)
    if lin1:
        xr = xj_ref[...].astype(BF)
        h = jnp.maximum(
            jnp.dot(xr, lw_ref[...], preferred_element_type=jnp.float32)
            + lb_ref[...], 0.0)
        xj = h.astype(BF)
    else:
        xj = xj_ref[...].astype(BF)
    parts = []
    for k in range(DE):
        bc = jnp.broadcast_to(ea_ref[:, k:k + 1].astype(BF), (TE, DF))
        parts.append(bc * xj)
    parts.append(xj)
    z = jnp.concatenate(parts, axis=1)
    acc = jnp.dot(z, w_ref[...], preferred_element_type=jnp.float32)
    rows = half * E_HALF + i * TE + lax.broadcasted_iota(jnp.int32, (TE, 1), 0)
    acc = jnp.where(rows < E, acc, 0.0)
    out_ref[...] = jnp.concatenate(
        [acc, jnp.zeros((TE, DF - dr), jnp.float32)], axis=1)


def _edge_mm(xj, ea, wbig, lw, lb, lin1, dr, half):
    def ea_map(i, h=half):
        return (i + h * NB_H, 0)
    return pl.pallas_call(
        functools.partial(_edge_mm_body, lin1, dr, half),
        grid=(NB_H,),
        in_specs=[
            pl.BlockSpec((TE, DF), lambda i: (i, 0)),
            pl.BlockSpec((TE, DE), ea_map),
            pl.BlockSpec(((DE + 1) * DF, dr), lambda i: (0, 0)),
            pl.BlockSpec((DF, DF), lambda i: (0, 0)),
            pl.BlockSpec((1, DF), lambda i: (0, 0)),
        ],
        out_specs=pl.BlockSpec((TE, DF), lambda i: (i, 0)),
        out_shape=jax.ShapeDtypeStruct((E_HALF, DF), jnp.float32),
    )(xj, ea, wbig, lw, lb)


def _combine1_body(aggA_ref, aggB_ref, hroot_ref, r2_ref, bs2_ref,
                   h2_ref, hroot2_ref):
    s = (aggA_ref[0][:, :H2] + aggA_ref[1][:, :H2]
         + aggB_ref[0][:, :H2] + aggB_ref[1][:, :H2] + hroot_ref[...])
    h2 = jnp.maximum(s, 0.0)
    h2_ref[...] = jnp.concatenate(
        [h2, jnp.zeros((BN, DF - H2), jnp.float32)], axis=1)
    hroot2_ref[...] = (
        jnp.dot(h2, r2_ref[...], preferred_element_type=jnp.float32) + bs2_ref[...])


def _final_body(agg2A_ref, agg2B_ref, hroot2_ref, out_ref):
    v = (agg2A_ref[0][:, :CP] + agg2A_ref[1][:, :CP]
         + agg2B_ref[0][:, :CP] + agg2B_ref[1][:, :CP] + hroot2_ref[...])
    col = lax.broadcasted_iota(jnp.int32, v.shape, 1)
    mask = col < C
    vm = jnp.where(mask, v, -1e30)
    m = jnp.max(vm, axis=1, keepdims=True)
    ex = jnp.where(mask, jnp.exp(v - m), 0.0)
    s = jnp.sum(ex, axis=1, keepdims=True)
    out_ref[...] = (v - m - jnp.log(s))[:, :C]


def kernel(x, edge_index, edge_attr, W1, b1, We1, be1, root1, bias1,
           We2, be2, root2, bias2):
    f32 = jnp.float32
    src_p = jnp.zeros((E_PAD,), jnp.int32).at[:E].set(edge_index[0])
    dst_p = jnp.zeros((E_PAD,), jnp.int32).at[:E].set(edge_index[1])
    srcA, srcB = src_p[:E_HALF], src_p[E_HALF:]
    dstA, dstB = dst_p[:E_HALF], dst_p[E_HALF:]
    ea_f = edge_attr

    wbig1 = jnp.concatenate(
        [We1.reshape(DE * H1, H2), be1.reshape(H1, H2)], axis=0).astype(BF)
    w2r = jnp.pad(We2.reshape(DE, H2, C), ((0, 0), (0, DF - H2), (0, CP - C)))
    be2r = jnp.pad(be2.reshape(H2, C), ((0, DF - H2), (0, CP - C)))
    wbig2 = jnp.concatenate(
        [w2r.reshape(DE * DF, CP), be2r], axis=0).astype(BF)
    w1_bf = W1.astype(BF)
    b1_row = b1.reshape(1, H1)
    root2p = jnp.pad(root2, ((0, 0), (0, CP - C)))
    bias2p = jnp.pad(bias2, (0, CP - C)).reshape(1, CP)
    zeros_acc = jnp.zeros((N_PAD, DF), f32)

    grid_n = (N // BN,)
    hroot = pl.pallas_call(
        _stage_a_body,
        grid=grid_n,
        in_specs=[
            pl.BlockSpec((BN, DF), lambda i: (i, 0)),
            pl.BlockSpec((DF, H1), lambda i: (0, 0)),
            pl.BlockSpec((1, H1), lambda i: (0, 0)),
            pl.BlockSpec((H1, H2), lambda i: (0, 0)),
            pl.BlockSpec((1, H2), lambda i: (0, 0)),
        ],
        out_specs=pl.BlockSpec((BN, H2), lambda i: (i, 0)),
        out_shape=jax.ShapeDtypeStruct((N, H2), f32),
    )(x, W1, b1_row, root1, bias1.reshape(1, H2))

    xrA = _sc_gather(x, srcA)
    xrB = _sc_gather(x, srcB)
    msgA = _edge_mm(xrA, ea_f, wbig1, w1_bf, b1_row, True, H2, 0)
    msgB = _edge_mm(xrB, ea_f, wbig1, w1_bf, b1_row, True, H2, 1)
    aggA = _sc_scatter(msgA, dstA, zeros_acc)
    aggB = _sc_scatter(msgB, dstB, zeros_acc)

    h2, hroot2 = pl.pallas_call(
        _combine1_body,
        grid=grid_n,
        in_specs=[
            pl.BlockSpec((NC, BN, DF), lambda i: (0, i, 0)),
            pl.BlockSpec((NC, BN, DF), lambda i: (0, i, 0)),
            pl.BlockSpec((BN, H2), lambda i: (i, 0)),
            pl.BlockSpec((H2, CP), lambda i: (0, 0)),
            pl.BlockSpec((1, CP), lambda i: (0, 0)),
        ],
        out_specs=[
            pl.BlockSpec((BN, DF), lambda i: (i, 0)),
            pl.BlockSpec((BN, CP), lambda i: (i, 0)),
        ],
        out_shape=[
            jax.ShapeDtypeStruct((N, DF), f32),
            jax.ShapeDtypeStruct((N, CP), f32),
        ],
    )(aggA, aggB, hroot, root2p, bias2p)

    xj2A = _sc_gather(h2, srcA)
    xj2B = _sc_gather(h2, srcB)
    msg2A = _edge_mm(xj2A, ea_f, wbig2, w1_bf, b1_row, False, CP, 0)
    msg2B = _edge_mm(xj2B, ea_f, wbig2, w1_bf, b1_row, False, CP, 1)
    agg2A = _sc_scatter(msg2A, dstA, zeros_acc)
    agg2B = _sc_scatter(msg2B, dstB, zeros_acc)

    out = pl.pallas_call(
        _final_body,
        grid=grid_n,
        in_specs=[
            pl.BlockSpec((NC, BN, DF), lambda i: (0, i, 0)),
            pl.BlockSpec((NC, BN, DF), lambda i: (0, i, 0)),
            pl.BlockSpec((BN, CP), lambda i: (i, 0)),
        ],
        out_specs=pl.BlockSpec((BN, C), lambda i: (i, 0)),
        out_shape=jax.ShapeDtypeStruct((N, C), f32),
    )(agg2A, agg2B, hroot2)

    return out
